# baseline (speedup 1.0000x reference)
import jax
import jax.numpy as jnp
from jax import lax
from jax.experimental import pallas as pl
from jax.experimental.pallas import tpu as pltpu
from jax.experimental.pallas import tpu_sc as plsc

LEVEL_RES = (16, 32, 64)
N_FEATS = 4
B = 16
N_PTS = 65536
N_CELLS = sum(r ** 3 for r in LEVEL_RES)
N_PARAMS = N_CELLS * N_FEATS
CELL_OFF = (0, 16 ** 3, 16 ** 3 + 32 ** 3)
N_OUT = 3 * N_FEATS
ROWS_PER_SCENE = N_PARAMS // 8

NC, NS = 2, 16
N_WORKERS = NC * NS
PTS_PER_WORKER = B * N_PTS // N_WORKERS
BLK = 128
N_BLKS = PTS_PER_WORKER // BLK
N_GRP = BLK // 16

PTS_BYTES = BLK * 3 * 4
OUT_BYTES = BLK * N_OUT * 4
GATHER_BYTES = 2 * 8 * BLK * 8 * 4


def _mm_body(z_ref, w_ref, o_ref):
    o_ref[...] = jnp.dot(z_ref[...], w_ref[...],
                         preferred_element_type=jnp.float32)


def _grow(z, W_grower):
    BN = 8192
    return pl.pallas_call(
        _mm_body,
        grid=(N_PARAMS // BN,),
        in_specs=[
            pl.BlockSpec((B, z.shape[1]), lambda i: (0, 0)),
            pl.BlockSpec((z.shape[1], BN), lambda i: (0, i)),
        ],
        out_specs=pl.BlockSpec((B, BN), lambda i: (0, i)),
        out_shape=jax.ShapeDtypeStruct((B, N_PARAMS), jnp.float32),
        compiler_params=pltpu.CompilerParams(
            dimension_semantics=("arbitrary",)),
    )(z, W_grower)


def _interp_body(pts_hbm, lodf_hbm, out_hbm, t3_hbm,
                 grid0, t1dA, t1dB, t2dA, t2dB, ptsA, ptsB, outA, outB,
                 i1a, w1a, p1a, i2a, w2a, p2a, r1a, r2a, i0a, w0a,
                 i1b, w1b, p1b, i2b, w2b, p2b, r1b, r2b, i0b, w0b,
                 sem_p, sem_ga, sem_gb, sem_oa, sem_ob,
                 sem_ri, sem_roA, sem_roB):
    c = lax.axis_index("c")
    s = lax.axis_index("s")
    b = c * (B // NC) + s // 2
    half = s % 2
    base_pt = half * PTS_PER_WORKER

    SA = (ptsA, outA, i1a, w1a, p1a, i2a, w2a, p2a, r1a, r2a, i0a, w0a)
    SB = (ptsB, outB, i1b, w1b, p1b, i2b, w2b, p2b, r1b, r2b, i0b, w0b)

    iota = lax.iota(jnp.int32, 16)

    iota_hi = jnp.right_shift(iota, 3)
    iota_lo = jnp.bitwise_and(iota, 7)
    HALF_ELEMS = N_PARAMS // 2
    HALF_ROWS = ROWS_PER_SCENE // 2
    N_CH = HALF_ELEMS // 8192

    def rp_in(ch, buf):
        return pltpu.async_copy(
            lodf_hbm.at[b, pl.ds(half * HALF_ELEMS + ch * 8192, 8192)],
            buf, sem_ri)

    def rp_wait_in(buf):
        pltpu.make_async_copy(
            lodf_hbm.at[0, pl.ds(0, 8192)], buf, sem_ri).wait()

    def rp_copy(t1d, t2d):
        def cp(kk, carry):
            base = kk * 256
            rbase = kk * 32
            for k16 in range(16):
                v = t1d[pl.ds(base + 16 * k16, 16)]
                plsc.store_scatter(
                    t2d, [iota_hi + (rbase + 2 * k16), iota_lo], v)
            return carry
        lax.fori_loop(0, 32, cp, 0)

    def rp_out(ch, buf, sem):
        return pltpu.async_copy(
            buf, t3_hbm.at[b, pl.ds(half * HALF_ROWS + ch * 1024, 1024), :],
            sem)

    def rp_wait_out(buf, sem):
        pltpu.make_async_copy(
            buf, t3_hbm.at[0, pl.ds(0, 1024), :], sem).wait()

    rp_in(0, t1dA)

    def repack(ch2, carry):
        c0 = 2 * ch2
        rp_wait_in(t1dA)
        rp_in(c0 + 1, t1dB)

        @pl.when(ch2 > 0)
        def _():
            rp_wait_out(t2dA, sem_roA)
        rp_copy(t1dA, t2dA)
        rp_out(c0, t2dA, sem_roA)

        rp_wait_in(t1dB)

        @pl.when(ch2 < (N_CH - 1) // 2 - 1)
        def _():
            rp_in(c0 + 2, t1dA)

        @pl.when(ch2 > 0)
        def _():
            rp_wait_out(t2dB, sem_roB)
        rp_copy(t1dB, t2dB)
        rp_out(c0 + 1, t2dB, sem_roB)
        return carry

    lax.fori_loop(0, (N_CH - 1) // 2, repack, 0)
    pltpu.sync_copy(
        lodf_hbm.at[b, pl.ds(half * HALF_ELEMS + (N_CH - 1) * 8192, 8192)],
        t1dA)
    rp_wait_out(t2dA, sem_roA)
    rp_copy(t1dA, t2dA)
    pltpu.sync_copy(
        t2dA,
        t3_hbm.at[b, pl.ds(half * HALF_ROWS + (N_CH - 1) * 1024, 1024), :])
    rp_wait_out(t2dB, sem_roB)
    plsc.subcore_barrier()
    lod8 = t3_hbm.at[b]

    pltpu.sync_copy(lodf_hbm.at[b, pl.ds(0, CELL_OFF[1] * N_FEATS)], grid0)

    def pts_dma(blk, buf):
        return pltpu.async_copy(
            pts_hbm.at[b, pl.ds((base_pt + blk * BLK) * 3, BLK * 3)],
            buf, sem_p)

    def phase_a(blk, S):
        pts_v, out_v, i1, w1, p1, i2, w2, p2 = S[:8]
        i0, w0 = S[10], S[11]
        for g in range(N_GRP):
            pvec = iota + (g * 16)
            pv3 = pvec * 3
            co = []
            for d in range(3):
                xr = plsc.load_gather(pts_v, [pv3 + d])
                co.append(jnp.clip(xr * 0.5 + 0.5, 0.0, 1.0))
            for lvl, R in enumerate(LEVEL_RES):
                x0i, w, u = [], [], []
                for d in range(3):
                    xs = co[d] * float(R - 1)
                    xi = jnp.clip(xs.astype(jnp.int32), 0, R - 2)
                    wd = jnp.clip(xs - xi.astype(jnp.float32), 0.0, 1.0)
                    x0i.append(xi)
                    w.append(wd)
                    u.append(1.0 - wd)
                basei = (x0i[0] * R + x0i[1]) * R + x0i[2] + CELL_OFF[lvl]
                wxy = [(w[0] if dx else u[0]) * (w[1] if dy else u[1])
                       for dx in (0, 1) for dy in (0, 1)]
                if lvl == 0:
                    i0[pl.ds(g * 16, 16)] = basei * 4
                    for ci, (dx, dy, dz) in enumerate(
                            (dx, dy, dz) for dx in (0, 1) for dy in (0, 1)
                            for dz in (0, 1)):
                        wc = wxy[dx * 2 + dy] * (w[2] if dz else u[2])
                        w0[pl.ds(ci * BLK + g * 16, 16)] = wc
                else:
                    iref, wref, pref = ((i1, w1, p1) if lvl == 1
                                        else (i2, w2, p2))
                    for j, (dx, dy) in enumerate(
                            (dx, dy) for dx in (0, 1) for dy in (0, 1)):
                        c0 = basei + (dx * R * R + dy * R)
                        r0 = jnp.right_shift(c0, 1)
                        iref[pl.ds(2 * j * BLK + g * 16, 16)] = r0
                        iref[pl.ds((2 * j + 1) * BLK + g * 16, 16)] = r0 + 1
                        pref[pl.ds(j * BLK + g * 16, 16)] = (
                            jnp.left_shift(jnp.bitwise_and(c0, 1), 2))
                        for dz in (0, 1):
                            wc = wxy[dx * 2 + dy] * (w[2] if dz else u[2])
                            wref[pl.ds((2 * j + dz) * BLK + g * 16, 16)] = wc

    def fire(S, sem):
        i1, i2, r1, r2 = S[2], S[5], S[8], S[9]
        pltpu.async_copy(lod8.at[i1], r1, sem)
        pltpu.async_copy(lod8.at[i2], r2, sem)

    def phase_c(S):
        _, out_v, _, w1, p1, _, w2, p2, r1, r2, i0, w0 = S
        for g in range(N_GRP):
            pvec = iota + (g * 16)
            base4 = i0[pl.ds(g * 16, 16)]
            acc = [jnp.zeros((16,), jnp.float32) for _ in range(4)]
            for ci, (dx, dy, dz) in enumerate(
                    (dx, dy, dz) for dx in (0, 1) for dy in (0, 1)
                    for dz in (0, 1)):
                wc = w0[pl.ds(ci * BLK + g * 16, 16)]
                idx4 = base4 + ((dx * 16 * 16 + dy * 16 + dz) * 4)
                for f in range(4):
                    v = plsc.load_gather(grid0, [idx4 + f])
                    acc[f] = acc[f] + v * wc
            for f in range(4):
                plsc.store_scatter(
                    out_v, [jnp.full((16,), f, jnp.int32), pvec], acc[f])
            for lvl, rows, wref, pref in ((1, r1, w1, p1), (2, r2, w2, p2)):
                acc = [jnp.zeros((16,), jnp.float32) for _ in range(4)]
                for j in range(4):
                    par4 = pref[pl.ds(j * BLK + g * 16, 16)]
                    for dz in (0, 1):
                        wc = wref[pl.ds((2 * j + dz) * BLK + g * 16, 16)]
                        t0 = par4 + (4 * dz)
                        rowv = pvec + (jnp.left_shift(jnp.right_shift(t0, 3), 7)
                                       + 2 * j * BLK)
                        mb = jnp.bitwise_and(t0, 7)
                        for f in range(4):
                            v = plsc.load_gather(rows, [rowv, mb + f])
                            acc[f] = acc[f] + v * wc
                for f in range(4):
                    plsc.store_scatter(
                        out_v,
                        [jnp.full((16,), 4 * lvl + f, jnp.int32), pvec],
                        acc[f])

    def out_dma(blk, buf, sem):
        return pltpu.async_copy(
            buf, out_hbm.at[pl.ds(b * N_OUT, N_OUT),
                            pl.ds(base_pt + blk * BLK, BLK)], sem)

    def wait_pts(buf):
        pltpu.make_async_copy(
            pts_hbm.at[0, pl.ds(0, BLK * 3)], buf, sem_p).wait()

    def wait_out(buf, sem):
        pltpu.make_async_copy(
            buf, out_hbm.at[pl.ds(0, N_OUT), pl.ds(0, BLK)], sem).wait()

    def wait_gather(S, sem):
        r1, r2 = S[8], S[9]
        pltpu.make_async_copy(lod8.at[pl.ds(0, 8 * BLK), :], r1, sem).wait()
        pltpu.make_async_copy(lod8.at[pl.ds(0, 8 * BLK), :], r2, sem).wait()

    pts_dma(0, ptsA).wait()
    phase_a(0, SA)
    fire(SA, sem_ga)
    pts_dma(1, ptsB)
    pts_dma(2, ptsA)

    def body(i2, carry):
        b0 = 2 * i2

        @pl.when(i2 > 0)
        def _():
            wait_out(outB, sem_ob)
        wait_pts(ptsB)
        phase_a(b0 + 1, SB)

        @pl.when(i2 < N_BLKS // 2 - 1)
        def _():
            pts_dma(b0 + 3, ptsB)
        fire(SB, sem_gb)

        wait_gather(SA, sem_ga)
        phase_c(SA)
        out_dma(b0, outA, sem_oa)

        @pl.when(i2 < N_BLKS // 2 - 1)
        def _():
            wait_out(outA, sem_oa)
            wait_pts(ptsA)
            phase_a(b0 + 2, SA)

            @pl.when(i2 < N_BLKS // 2 - 2)
            def _():
                pts_dma(b0 + 4, ptsA)
            fire(SA, sem_ga)

        wait_gather(SB, sem_gb)
        phase_c(SB)
        out_dma(b0 + 1, outB, sem_ob)
        return carry

    lax.fori_loop(0, N_BLKS // 2, body, 0)
    wait_out(outA, sem_oa)
    wait_out(outB, sem_ob)


def _interp(pts_flat, lodf):
    mesh = plsc.VectorSubcoreMesh(core_axis_name="c", subcore_axis_name="s")
    lvl_bufs = lambda: [
        pltpu.VMEM((8 * BLK,), jnp.int32),
        pltpu.VMEM((8 * BLK,), jnp.float32),
        pltpu.VMEM((4 * BLK,), jnp.int32),
        pltpu.VMEM((8 * BLK,), jnp.int32),
        pltpu.VMEM((8 * BLK,), jnp.float32),
        pltpu.VMEM((4 * BLK,), jnp.int32),
        pltpu.VMEM((8 * BLK, 8), jnp.float32),
        pltpu.VMEM((8 * BLK, 8), jnp.float32),
        pltpu.VMEM((BLK,), jnp.int32),
        pltpu.VMEM((8 * BLK,), jnp.float32),
    ]
    f = pl.kernel(
        _interp_body,
        out_type=(jax.ShapeDtypeStruct((B * N_OUT, N_PTS), jnp.float32),
                  jax.ShapeDtypeStruct((B, ROWS_PER_SCENE, 8), jnp.float32)),
        mesh=mesh,
        scratch_types=[
            pltpu.VMEM((CELL_OFF[1] * N_FEATS,), jnp.float32),
            pltpu.VMEM((8192,), jnp.float32),
            pltpu.VMEM((8192,), jnp.float32),
            pltpu.VMEM((1024, 8), jnp.float32),
            pltpu.VMEM((1024, 8), jnp.float32),
            pltpu.VMEM((BLK * 3,), jnp.float32),
            pltpu.VMEM((BLK * 3,), jnp.float32),
            pltpu.VMEM((N_OUT, BLK), jnp.float32),
            pltpu.VMEM((N_OUT, BLK), jnp.float32),
            *lvl_bufs(),
            *lvl_bufs(),
            pltpu.SemaphoreType.DMA,
            pltpu.SemaphoreType.DMA,
            pltpu.SemaphoreType.DMA,
            pltpu.SemaphoreType.DMA,
            pltpu.SemaphoreType.DMA,
            pltpu.SemaphoreType.DMA,
            pltpu.SemaphoreType.DMA,
            pltpu.SemaphoreType.DMA,
        ],
        compiler_params=pltpu.CompilerParams(
            needs_layout_passes=False, use_tc_tiling_on_sc=False),
    )
    return f(pts_flat, lodf)[0]


@jax.jit
def kernel(input, z, W_grower):
    lod = _grow(z, W_grower)
    out = _interp(input.reshape(B, N_PTS * 3), lod)
    return jnp.transpose(out.reshape(B, N_OUT, N_PTS), (0, 2, 1))


# speedup vs baseline: 1.0443x; 1.0443x over previous
import jax
import jax.numpy as jnp
from jax import lax
from jax.experimental import pallas as pl
from jax.experimental.pallas import tpu as pltpu
from jax.experimental.pallas import tpu_sc as plsc

LEVEL_RES = (16, 32, 64)
N_FEATS = 4
B = 16
N_PTS = 65536
N_CELLS = sum(r ** 3 for r in LEVEL_RES)
N_PARAMS = N_CELLS * N_FEATS
CELL_OFF = (0, 16 ** 3, 16 ** 3 + 32 ** 3)
N_OUT = 3 * N_FEATS
ROWS_PER_SCENE = N_PARAMS // 8

NC, NS = 2, 16
N_WORKERS = NC * NS
PTS_PER_WORKER = B * N_PTS // N_WORKERS
BLK = 128
N_BLKS = PTS_PER_WORKER // BLK
N_GRP = BLK // 16

PTS_BYTES = BLK * 3 * 4
OUT_BYTES = BLK * N_OUT * 4
GATHER_BYTES = 2 * 8 * BLK * 8 * 4


def _mm_body(z_ref, w_ref, o_ref):
    o_ref[...] = jnp.dot(z_ref[...], w_ref[...],
                         preferred_element_type=jnp.float32)


def _grow(z, W_grower):
    BN = 16384
    return pl.pallas_call(
        _mm_body,
        grid=(N_PARAMS // BN,),
        in_specs=[
            pl.BlockSpec((B, z.shape[1]), lambda i: (0, 0)),
            pl.BlockSpec((z.shape[1], BN), lambda i: (0, i)),
        ],
        out_specs=pl.BlockSpec((B, BN), lambda i: (0, i)),
        out_shape=jax.ShapeDtypeStruct((B, N_PARAMS), jnp.float32),
        compiler_params=pltpu.CompilerParams(
            dimension_semantics=("arbitrary",)),
    )(z, W_grower)


def _interp_body(pts_hbm, lodf_hbm, out_hbm, t3_hbm,
                 grid0, t1dA, t1dB, t2dA, t2dB, ptsA, ptsB, outA, outB,
                 i1a, w1a, p1a, i2a, w2a, p2a, r1a, r2a,
                 i1b, w1b, p1b, i2b, w2b, p2b, r1b, r2b,
                 sem_p, sem_ga, sem_gb, sem_oa, sem_ob,
                 sem_ri, sem_roA, sem_roB):
    c = lax.axis_index("c")
    s = lax.axis_index("s")
    b = c * (B // NC) + s // 2
    half = s % 2
    base_pt = half * PTS_PER_WORKER

    SA = (ptsA, outA, i1a, w1a, p1a, i2a, w2a, p2a, r1a, r2a)
    SB = (ptsB, outB, i1b, w1b, p1b, i2b, w2b, p2b, r1b, r2b)

    iota = lax.iota(jnp.int32, 16)

    iota_hi = jnp.right_shift(iota, 3)
    iota_lo = jnp.bitwise_and(iota, 7)
    HALF_ELEMS = N_PARAMS // 2
    HALF_ROWS = ROWS_PER_SCENE // 2
    N_CH = HALF_ELEMS // 8192

    def rp_in(ch, buf):
        return pltpu.async_copy(
            lodf_hbm.at[b, pl.ds(half * HALF_ELEMS + ch * 8192, 8192)],
            buf, sem_ri)

    def rp_wait_in(buf):
        pltpu.make_async_copy(
            lodf_hbm.at[0, pl.ds(0, 8192)], buf, sem_ri).wait()

    def rp_copy(t1d, t2d):
        def cp(kk, carry):
            base = kk * 256
            rbase = kk * 32
            for k16 in range(16):
                v = t1d[pl.ds(base + 16 * k16, 16)]
                plsc.store_scatter(
                    t2d, [iota_hi + (rbase + 2 * k16), iota_lo], v)
            return carry
        lax.fori_loop(0, 32, cp, 0)

    def rp_out(ch, buf, sem):
        return pltpu.async_copy(
            buf, t3_hbm.at[b, pl.ds(half * HALF_ROWS + ch * 1024, 1024), :],
            sem)

    def rp_wait_out(buf, sem):
        pltpu.make_async_copy(
            buf, t3_hbm.at[0, pl.ds(0, 1024), :], sem).wait()

    rp_in(0, t1dA)

    def repack(ch2, carry):
        c0 = 2 * ch2
        rp_wait_in(t1dA)
        rp_in(c0 + 1, t1dB)

        @pl.when(ch2 > 0)
        def _():
            rp_wait_out(t2dA, sem_roA)
        rp_copy(t1dA, t2dA)
        rp_out(c0, t2dA, sem_roA)

        rp_wait_in(t1dB)

        @pl.when(ch2 < (N_CH - 1) // 2 - 1)
        def _():
            rp_in(c0 + 2, t1dA)

        @pl.when(ch2 > 0)
        def _():
            rp_wait_out(t2dB, sem_roB)
        rp_copy(t1dB, t2dB)
        rp_out(c0 + 1, t2dB, sem_roB)
        return carry

    lax.fori_loop(0, (N_CH - 1) // 2, repack, 0)
    pltpu.sync_copy(
        lodf_hbm.at[b, pl.ds(half * HALF_ELEMS + (N_CH - 1) * 8192, 8192)],
        t1dA)
    rp_wait_out(t2dA, sem_roA)
    rp_copy(t1dA, t2dA)
    pltpu.sync_copy(
        t2dA,
        t3_hbm.at[b, pl.ds(half * HALF_ROWS + (N_CH - 1) * 1024, 1024), :])
    rp_wait_out(t2dB, sem_roB)
    plsc.subcore_barrier()
    lod8 = t3_hbm.at[b]

    pltpu.sync_copy(lodf_hbm.at[b, pl.ds(0, CELL_OFF[1] * N_FEATS)], grid0)

    def pts_dma(blk, buf):
        return pltpu.async_copy(
            pts_hbm.at[b, pl.ds((base_pt + blk * BLK) * 3, BLK * 3)],
            buf, sem_p)

    def phase_a(blk, S):
        pts_v, out_v, i1, w1, p1, i2, w2, p2 = S[:8]
        for g in range(N_GRP):
            pvec = iota + (g * 16)
            pv3 = pvec * 3
            co = []
            for d in range(3):
                xr = plsc.load_gather(pts_v, [pv3 + d])
                co.append(jnp.clip(xr * 0.5 + 0.5, 0.0, 1.0))
            for lvl, R in enumerate(LEVEL_RES):
                x0i, w, u = [], [], []
                for d in range(3):
                    xs = co[d] * float(R - 1)
                    xi = jnp.clip(xs.astype(jnp.int32), 0, R - 2)
                    wd = jnp.clip(xs - xi.astype(jnp.float32), 0.0, 1.0)
                    x0i.append(xi)
                    w.append(wd)
                    u.append(1.0 - wd)
                basei = (x0i[0] * R + x0i[1]) * R + x0i[2] + CELL_OFF[lvl]
                wxy = [(w[0] if dx else u[0]) * (w[1] if dy else u[1])
                       for dx in (0, 1) for dy in (0, 1)]
                if lvl == 0:
                    base4 = basei * 4
                    acc = [jnp.zeros((16,), jnp.float32) for _ in range(4)]
                    for ci, (dx, dy, dz) in enumerate(
                            (dx, dy, dz) for dx in (0, 1) for dy in (0, 1)
                            for dz in (0, 1)):
                        wc = wxy[dx * 2 + dy] * (w[2] if dz else u[2])
                        idx4 = base4 + ((dx * 16 * 16 + dy * 16 + dz) * 4)
                        for f in range(4):
                            v = plsc.load_gather(grid0, [idx4 + f])
                            acc[f] = acc[f] + v * wc
                    for f in range(4):
                        plsc.store_scatter(
                            out_v, [jnp.full((16,), f, jnp.int32), pvec],
                            acc[f])
                else:
                    iref, wref, pref = ((i1, w1, p1) if lvl == 1
                                        else (i2, w2, p2))
                    for j, (dx, dy) in enumerate(
                            (dx, dy) for dx in (0, 1) for dy in (0, 1)):
                        c0 = basei + (dx * R * R + dy * R)
                        r0 = jnp.right_shift(c0, 1)
                        iref[pl.ds(2 * j * BLK + g * 16, 16)] = r0
                        iref[pl.ds((2 * j + 1) * BLK + g * 16, 16)] = r0 + 1
                        pref[pl.ds(j * BLK + g * 16, 16)] = (
                            jnp.left_shift(jnp.bitwise_and(c0, 1), 2))
                        for dz in (0, 1):
                            wc = wxy[dx * 2 + dy] * (w[2] if dz else u[2])
                            wref[pl.ds((2 * j + dz) * BLK + g * 16, 16)] = wc

    def fire(S, sem):
        _, _, i1, _, _, i2, _, _, r1, r2 = S
        pltpu.async_copy(lod8.at[i1], r1, sem)
        pltpu.async_copy(lod8.at[i2], r2, sem)

    def phase_c(S):
        _, out_v, _, w1, p1, _, w2, p2, r1, r2 = S
        for g in range(N_GRP):
            pvec = iota + (g * 16)
            for lvl, rows, wref, pref in ((1, r1, w1, p1), (2, r2, w2, p2)):
                acc = [jnp.zeros((16,), jnp.float32) for _ in range(4)]
                for j in range(4):
                    par4 = pref[pl.ds(j * BLK + g * 16, 16)]
                    for dz in (0, 1):
                        wc = wref[pl.ds((2 * j + dz) * BLK + g * 16, 16)]
                        t0 = par4 + (4 * dz)
                        rowv = pvec + (jnp.left_shift(jnp.right_shift(t0, 3), 7)
                                       + 2 * j * BLK)
                        mb = jnp.bitwise_and(t0, 7)
                        for f in range(4):
                            v = plsc.load_gather(rows, [rowv, mb + f])
                            acc[f] = acc[f] + v * wc
                for f in range(4):
                    plsc.store_scatter(
                        out_v,
                        [jnp.full((16,), 4 * lvl + f, jnp.int32), pvec],
                        acc[f])

    def out_dma(blk, buf, sem):
        return pltpu.async_copy(
            buf, out_hbm.at[pl.ds(b * N_OUT, N_OUT),
                            pl.ds(base_pt + blk * BLK, BLK)], sem)

    def wait_pts(buf):
        pltpu.make_async_copy(
            pts_hbm.at[0, pl.ds(0, BLK * 3)], buf, sem_p).wait()

    def wait_out(buf, sem):
        pltpu.make_async_copy(
            buf, out_hbm.at[pl.ds(0, N_OUT), pl.ds(0, BLK)], sem).wait()

    def wait_gather(S, sem):
        _, _, _, _, _, _, _, _, r1, r2 = S
        pltpu.make_async_copy(lod8.at[pl.ds(0, 8 * BLK), :], r1, sem).wait()
        pltpu.make_async_copy(lod8.at[pl.ds(0, 8 * BLK), :], r2, sem).wait()

    pts_dma(0, ptsA).wait()
    phase_a(0, SA)
    fire(SA, sem_ga)
    pts_dma(1, ptsB)
    pts_dma(2, ptsA)

    def body(i2, carry):
        b0 = 2 * i2

        @pl.when(i2 > 0)
        def _():
            wait_out(outB, sem_ob)
        wait_pts(ptsB)
        phase_a(b0 + 1, SB)

        @pl.when(i2 < N_BLKS // 2 - 1)
        def _():
            pts_dma(b0 + 3, ptsB)
        fire(SB, sem_gb)

        wait_gather(SA, sem_ga)
        phase_c(SA)
        out_dma(b0, outA, sem_oa)

        @pl.when(i2 < N_BLKS // 2 - 1)
        def _():
            wait_out(outA, sem_oa)
            wait_pts(ptsA)
            phase_a(b0 + 2, SA)

            @pl.when(i2 < N_BLKS // 2 - 2)
            def _():
                pts_dma(b0 + 4, ptsA)
            fire(SA, sem_ga)

        wait_gather(SB, sem_gb)
        phase_c(SB)
        out_dma(b0 + 1, outB, sem_ob)
        return carry

    lax.fori_loop(0, N_BLKS // 2, body, 0)
    wait_out(outA, sem_oa)
    wait_out(outB, sem_ob)


def _interp(pts_flat, lodf):
    mesh = plsc.VectorSubcoreMesh(core_axis_name="c", subcore_axis_name="s")
    lvl_bufs = lambda: [
        pltpu.VMEM((8 * BLK,), jnp.int32),
        pltpu.VMEM((8 * BLK,), jnp.float32),
        pltpu.VMEM((4 * BLK,), jnp.int32),
        pltpu.VMEM((8 * BLK,), jnp.int32),
        pltpu.VMEM((8 * BLK,), jnp.float32),
        pltpu.VMEM((4 * BLK,), jnp.int32),
        pltpu.VMEM((8 * BLK, 8), jnp.float32),
        pltpu.VMEM((8 * BLK, 8), jnp.float32),
    ]
    f = pl.kernel(
        _interp_body,
        out_type=(jax.ShapeDtypeStruct((B * N_OUT, N_PTS), jnp.float32),
                  jax.ShapeDtypeStruct((B, ROWS_PER_SCENE, 8), jnp.float32)),
        mesh=mesh,
        scratch_types=[
            pltpu.VMEM((CELL_OFF[1] * N_FEATS,), jnp.float32),
            pltpu.VMEM((8192,), jnp.float32),
            pltpu.VMEM((8192,), jnp.float32),
            pltpu.VMEM((1024, 8), jnp.float32),
            pltpu.VMEM((1024, 8), jnp.float32),
            pltpu.VMEM((BLK * 3,), jnp.float32),
            pltpu.VMEM((BLK * 3,), jnp.float32),
            pltpu.VMEM((N_OUT, BLK), jnp.float32),
            pltpu.VMEM((N_OUT, BLK), jnp.float32),
            *lvl_bufs(),
            *lvl_bufs(),
            pltpu.SemaphoreType.DMA,
            pltpu.SemaphoreType.DMA,
            pltpu.SemaphoreType.DMA,
            pltpu.SemaphoreType.DMA,
            pltpu.SemaphoreType.DMA,
            pltpu.SemaphoreType.DMA,
            pltpu.SemaphoreType.DMA,
            pltpu.SemaphoreType.DMA,
        ],
        compiler_params=pltpu.CompilerParams(
            needs_layout_passes=False, use_tc_tiling_on_sc=False),
    )
    return f(pts_flat, lodf)[0]


@jax.jit
def kernel(input, z, W_grower):
    lod = _grow(z, W_grower)
    out = _interp(input.reshape(B, N_PTS * 3), lod)
    return jnp.transpose(out.reshape(B, N_OUT, N_PTS), (0, 2, 1))
